# Initial kernel scaffold; baseline (speedup 1.0000x reference)
import jax
import jax.numpy as jnp
from jax.experimental import pallas as pl


def kernel(query, value, reference_points, spatial_shapes, W_so, b_so, W_aw, b_aw, W_v, b_v, W_o, b_o):
    raise NotImplementedError("write your pallas kernel here")


# speedup vs baseline: 57.3721x; 57.3721x over previous
import functools

import jax
import jax.numpy as jnp
import numpy as np
from jax import lax
from jax.experimental import pallas as pl
from jax.experimental.pallas import tpu as pltpu
from jax.experimental.pallas import tpu_sc as plsc

_NH, _NL, _NPT, _D = 8, 4, 4, 256
_DH = _D // _NH
_SHAPES = [(96, 96), (48, 48), (24, 24), (12, 12)]
_B = 2
_V = sum(h * w for h, w in _SHAPES)
_Q = _V
_NC = 128

_QB = 240
_NQB = _Q // _QB

_NW = 32
_QPW = _Q // 16
_NQ = 5
_NBLK = _QPW // _NQ
_NROW = _NQ * 4 * _NC
_NGATH = _NROW // 128

_CB = 2040
_NCB = _B * _Q // _CB

_cols = np.arange(_NC)
_lvl = (_cols // 4) % _NL
_scol_np = np.array([s[0] for s in _SHAPES], np.float32)[_lvl]
_start_np = np.array([0, 9216, 11520, 12096], np.int32)[_lvl]
_hcol_np = (_cols // 16).astype(np.int32)
_Ex_np = np.zeros((8, _NC), np.float32)
_Ey_np = np.zeros((8, _NC), np.float32)
for _c in range(_NC):
    _Ex_np[2 * _lvl[_c], _c] = 1.0
    _Ey_np[2 * _lvl[_c] + 1, _c] = 1.0
_S_np = (np.arange(_NC)[:, None] // 16 == np.arange(_NC)[None, :] // 16).astype(np.float32)


def _stage_a(q_ref, val_ref, ref8_ref, wsox_ref, wsoy_ref, bsox_ref, bsoy_ref,
             waw_ref, baw_ref, wv_ref, bv_ref, ex_ref, ey_ref, scol_ref,
             scoli_ref, start_ref, hcol_ref, sdiag_ref,
             idx_ref, w_ref, vout_ref):
    b = pl.program_id(0)
    q = q_ref[0]
    f32 = jnp.float32
    sox = jnp.dot(q, wsox_ref[...], preferred_element_type=f32, precision=jax.lax.Precision.HIGHEST) + bsox_ref[...]
    soy = jnp.dot(q, wsoy_ref[...], preferred_element_type=f32, precision=jax.lax.Precision.HIGHEST) + bsoy_ref[...]
    logits = jnp.dot(q, waw_ref[...], preferred_element_type=f32, precision=jax.lax.Precision.HIGHEST) + baw_ref[...]
    e = jnp.exp(logits)
    denom = jnp.dot(e, sdiag_ref[...], preferred_element_type=f32, precision=jax.lax.Precision.HIGHEST)
    aw = e / denom
    ref8 = ref8_ref[0]
    refx = jnp.dot(ref8, ex_ref[...], preferred_element_type=f32, precision=jax.lax.Precision.HIGHEST)
    refy = jnp.dot(ref8, ey_ref[...], preferred_element_type=f32, precision=jax.lax.Precision.HIGHEST)
    scol = scol_ref[...]
    gridx = 2.0 * (refx + sox / scol) - 1.0
    gridy = 2.0 * (refy + soy / scol) - 1.0
    gx = (gridx + 1.0) * scol / 2.0 - 0.5
    gy = (gridy + 1.0) * scol / 2.0 - 0.5
    x0 = jnp.floor(gx)
    y0 = jnp.floor(gy)
    fx = gx - x0
    fy = gy - y0
    smax = scol - 1.0
    scoli = scoli_ref[...]
    starti = start_ref[...]
    hcol = hcol_ref[...]
    base = b * _V

    def corner(xf, yf, wgt):
        ok = (xf >= 0.0) & (xf <= smax) & (yf >= 0.0) & (yf <= smax)
        xc = jnp.clip(xf, 0.0, smax).astype(jnp.int32)
        yc = jnp.clip(yf, 0.0, smax).astype(jnp.int32)
        row = (yc * scoli + xc + starti + base) * _NH + hcol
        return row, aw * wgt * ok.astype(f32)

    r00, w00 = corner(x0, y0, (1.0 - fx) * (1.0 - fy))
    r10, w10 = corner(x0 + 1.0, y0, fx * (1.0 - fy))
    r01, w01 = corner(x0, y0 + 1.0, (1.0 - fx) * fy)
    r11, w11 = corner(x0 + 1.0, y0 + 1.0, fx * fy)
    idx_ref[0, 0] = r00
    idx_ref[0, 1] = r10
    idx_ref[0, 2] = r01
    idx_ref[0, 3] = r11
    w_ref[0, 0] = w00
    w_ref[0, 1] = w10
    w_ref[0, 2] = w01
    w_ref[0, 3] = w11
    vout_ref[0] = jnp.dot(val_ref[0], wv_ref[...], preferred_element_type=f32, precision=jax.lax.Precision.HIGHEST) + bv_ref[...]


def _stage_c(s_ref, wo_ref, bo_ref, o_ref):
    o_ref[...] = jnp.dot(s_ref[...], wo_ref[...],
                         preferred_element_type=jnp.float32,
                         precision=jax.lax.Precision.HIGHEST) + bo_ref[...]


def _sc_sample(table, idx, wgt):
    mesh = plsc.VectorSubcoreMesh(core_axis_name="c", subcore_axis_name="s")

    @functools.partial(
        pl.kernel,
        out_type=jax.ShapeDtypeStruct((_B * _Q * _D,), jnp.float32),
        mesh=mesh,
        scratch_types=[
            pltpu.VMEM((_NROW,), jnp.int32),
            pltpu.VMEM((_NROW,), jnp.float32),
            pltpu.VMEM((_NROW, _DH), jnp.float32),
            pltpu.VMEM((_NQ * _D,), jnp.float32),
            pltpu.SemaphoreType.DMA,
        ],
        compiler_params=pltpu.CompilerParams(needs_layout_passes=False,
                                             use_tc_tiling_on_sc=False),
    )
    def run(table_h, idx_h, w_h, out_h, idx_v, w_v, rows_v, out_v, sem):
        cid = lax.axis_index("c")
        sid = lax.axis_index("s")
        iota = lax.broadcasted_iota(jnp.int32, (16,), 0)
        nseg = _NQ * _NC

        def blk_body(blk, carry):
            qb = sid * _QPW + blk * _NQ
            src0 = (cid * 4 * _Q + qb) * _NC
            for cnr in range(4):
                pltpu.sync_copy(idx_h.at[pl.ds(src0 + cnr * _Q * _NC, nseg)],
                                idx_v.at[pl.ds(cnr * nseg, nseg)])
                pltpu.sync_copy(w_h.at[pl.ds(src0 + cnr * _Q * _NC, nseg)],
                                w_v.at[pl.ds(cnr * nseg, nseg)])
            descs = [
                pltpu.async_copy(table_h.at[idx_v.at[pl.ds(j * 128, 128)]],
                                 rows_v.at[pl.ds(j * 128, 128), :], sem)
                for j in range(_NGATH)
            ]
            for d in descs:
                d.wait()

            def qh_body(qh, c2):
                qq = qh // 8
                h = qh % 8

                def e_body(k, acc):
                    a0, a1 = acc
                    cnr = k // 16
                    j = k % 16
                    rr = (cnr * _NQ + qq) * _NC + h * 16 + j
                    ws = plsc.load_gather(w_v, [jnp.full((16,), rr, jnp.int32)])
                    rfull = jnp.full((16,), rr, jnp.int32)
                    lo = plsc.load_gather(rows_v, [rfull, iota])
                    hi = plsc.load_gather(rows_v, [rfull, iota + 16])
                    return (a0 + ws * lo, a1 + ws * hi)

                z = jnp.zeros((16,), jnp.float32)
                a0, a1 = lax.fori_loop(0, 64, e_body, (z, z))
                obase = qq * _D + h * 32
                plsc.store_scatter(out_v, [obase + iota], a0)
                plsc.store_scatter(out_v, [obase + 16 + iota], a1)
                return c2

            lax.fori_loop(0, _NQ * 8, qh_body, 0)
            pltpu.sync_copy(out_v, out_h.at[pl.ds((cid * _Q + qb) * _D, _NQ * _D)])
            return carry

        lax.fori_loop(0, _NBLK, blk_body, 0)

    return run(table, idx.reshape(-1), wgt.reshape(-1))


def _stage_a_call(query, value, reference_points, W_so, b_so, W_aw, b_aw,
                  W_v, b_v):
    f32 = jnp.float32
    W_sox = W_so.reshape(_D, _NH, _NL, _NPT, 2)[..., 0].reshape(_D, _NC)
    W_soy = W_so.reshape(_D, _NH, _NL, _NPT, 2)[..., 1].reshape(_D, _NC)
    b_sox = b_so.reshape(_NH, _NL, _NPT, 2)[..., 0].reshape(1, _NC)
    b_soy = b_so.reshape(_NH, _NL, _NPT, 2)[..., 1].reshape(1, _NC)
    baw2 = b_aw.reshape(1, _NC)
    bv2 = b_v.reshape(1, _D)
    ref8 = reference_points.reshape(_B, _Q, _NL * 2)
    scolf = jnp.asarray(_scol_np.reshape(1, _NC))
    scoli = jnp.asarray(_scol_np.astype(np.int32).reshape(1, _NC))
    starti = jnp.asarray(_start_np.reshape(1, _NC))
    hcoli = jnp.asarray(_hcol_np.reshape(1, _NC))
    Ex = jnp.asarray(_Ex_np)
    Ey = jnp.asarray(_Ey_np)
    Sd = jnp.asarray(_S_np)

    full = lambda shp: pl.BlockSpec(shp, lambda b, i: tuple(0 for _ in shp))
    idx, wgt, vproj = pl.pallas_call(
        _stage_a,
        grid=(_B, _NQB),
        in_specs=[
            pl.BlockSpec((1, _QB, _D), lambda b, i: (b, i, 0)),
            pl.BlockSpec((1, _QB, _D), lambda b, i: (b, i, 0)),
            pl.BlockSpec((1, _QB, 8), lambda b, i: (b, i, 0)),
            full((_D, _NC)), full((_D, _NC)), full((1, _NC)), full((1, _NC)),
            full((_D, _NC)), full((1, _NC)), full((_D, _D)), full((1, _D)),
            full((8, _NC)), full((8, _NC)), full((1, _NC)), full((1, _NC)),
            full((1, _NC)), full((1, _NC)), full((_NC, _NC)),
        ],
        out_specs=[
            pl.BlockSpec((1, 4, _QB, _NC), lambda b, i: (b, 0, i, 0)),
            pl.BlockSpec((1, 4, _QB, _NC), lambda b, i: (b, 0, i, 0)),
            pl.BlockSpec((1, _QB, _D), lambda b, i: (b, i, 0)),
        ],
        out_shape=[
            jax.ShapeDtypeStruct((_B, 4, _Q, _NC), jnp.int32),
            jax.ShapeDtypeStruct((_B, 4, _Q, _NC), f32),
            jax.ShapeDtypeStruct((_B, _Q, _D), f32),
        ],
    )(query, value, ref8, W_sox, W_soy, b_sox, b_soy, W_aw, baw2, W_v, bv2,
      Ex, Ey, scolf, scoli, starti, hcoli, Sd)
    return idx, wgt, vproj


def kernel(query, value, reference_points, spatial_shapes, W_so, b_so,
           W_aw, b_aw, W_v, b_v, W_o, b_o):
    f32 = jnp.float32
    idx, wgt, vproj = _stage_a_call(query, value, reference_points,
                                    W_so, b_so, W_aw, b_aw, W_v, b_v)
    bo2 = b_o.reshape(1, _D)
    table = vproj.reshape(_B * _V * _NH, _DH)
    sampled = _sc_sample(table, idx, wgt).reshape(_B * _Q, _D)

    out = pl.pallas_call(
        _stage_c,
        grid=(_NCB,),
        in_specs=[
            pl.BlockSpec((_CB, _D), lambda i: (i, 0)),
            pl.BlockSpec((_D, _D), lambda i: (0, 0)),
            pl.BlockSpec((1, _D), lambda i: (0, 0)),
        ],
        out_specs=pl.BlockSpec((_CB, _D), lambda i: (i, 0)),
        out_shape=jax.ShapeDtypeStruct((_B * _Q, _D), f32),
    )(sampled, W_o, bo2)
    return out.reshape(_B, _Q, _D)
